# baseline (speedup 1.0000x reference)
import functools

import jax
import jax.numpy as jnp
from jax import lax
from jax.experimental import pallas as pl
from jax.experimental.pallas import tpu as pltpu
from jax.experimental.pallas import tpu_sc as plsc

_D = 32
_TT = 20
_MT = 50
_GR = 16
_NG = 4
_WIN = _GR * _NG

_NC, _NS = 2, 16
_NW = _NC * _NS
_CH = 128


def _sc_gather(user_table, item_table, utt_g, itt_g, ui2, ii2, ug2, ig2):
    b = ui2.shape[0] * _CH
    rows_w = b // _NW
    nch = rows_w // _CH
    gch = rows_w * _NG // _CH
    mesh = plsc.VectorSubcoreMesh(core_axis_name="c", subcore_axis_name="s")

    @functools.partial(
        pl.kernel,
        out_type=[
            jax.ShapeDtypeStruct((b, _D), jnp.float32),
            jax.ShapeDtypeStruct((b, _D), jnp.float32),
            jax.ShapeDtypeStruct((b * _NG, _GR), jnp.float32),
            jax.ShapeDtypeStruct((b * _NG, _GR), jnp.float32),
        ],
        mesh=mesh,
        scratch_types=[
            pltpu.VMEM((nch, _CH), jnp.int32),
            pltpu.VMEM((nch, _CH), jnp.int32),
            pltpu.VMEM((gch, _CH), jnp.int32),
            pltpu.VMEM((gch, _CH), jnp.int32),
            pltpu.VMEM((rows_w, _D), jnp.float32),
            pltpu.VMEM((rows_w, _D), jnp.float32),
            pltpu.VMEM((rows_w * _NG, _GR), jnp.float32),
            pltpu.VMEM((rows_w * _NG, _GR), jnp.float32),
            pltpu.SemaphoreType.DMA,
        ],
        compiler_params=pltpu.CompilerParams(use_tc_tiling_on_sc=False),
    )
    def k(ut_hbm, it_hbm, utt_hbm, itt_hbm, ui_hbm, ii_hbm, ug_hbm, ig_hbm,
          ue_hbm, ie_hbm, ute_hbm, ite_hbm,
          uidx_v, iidx_v, ugidx_v, igidx_v, ue_v, ie_v, ute_v, ite_v, sem):
        wid = lax.axis_index("s") * _NC + lax.axis_index("c")
        base = wid * rows_w
        pltpu.sync_copy(ui_hbm.at[pl.ds(wid * nch, nch)], uidx_v)
        pltpu.sync_copy(ii_hbm.at[pl.ds(wid * nch, nch)], iidx_v)
        pltpu.sync_copy(ug_hbm.at[pl.ds(wid * gch, gch)], ugidx_v)
        pltpu.sync_copy(ig_hbm.at[pl.ds(wid * gch, gch)], igidx_v)
        cps = []
        for j in range(nch):
            sl = pl.ds(j * _CH, _CH)
            cps.append(pltpu.async_copy(ut_hbm.at[uidx_v.at[j]], ue_v.at[sl], sem))
            cps.append(pltpu.async_copy(it_hbm.at[iidx_v.at[j]], ie_v.at[sl], sem))
        for j in range(gch):
            sl = pl.ds(j * _CH, _CH)
            cps.append(pltpu.async_copy(utt_hbm.at[ugidx_v.at[j]], ute_v.at[sl], sem))
            cps.append(pltpu.async_copy(itt_hbm.at[igidx_v.at[j]], ite_v.at[sl], sem))
        for cp in cps:
            cp.wait()
        pltpu.sync_copy(ue_v, ue_hbm.at[pl.ds(base, rows_w)])
        pltpu.sync_copy(ie_v, ie_hbm.at[pl.ds(base, rows_w)])
        pltpu.sync_copy(ute_v, ute_hbm.at[pl.ds(base * _NG, rows_w * _NG)])
        pltpu.sync_copy(ite_v, ite_hbm.at[pl.ds(base * _NG, rows_w * _NG)])

    ue, ie, uteg, iteg = k(user_table, item_table, utt_g, itt_g, ui2, ii2, ug2, ig2)
    return ue, ie, uteg.reshape(b, _WIN), iteg.reshape(b, _WIN)


def _realign(win, off):
    out = win[:, 0:_MT]
    for o in range(2, _GR, 2):
        out = jnp.where(off == o, win[:, o:o + _MT], out)
    return out


def _tc_body(ue, ie, uteg, iteg, uoff, ioff, utr, itr, w1, b1, w2, b2, gm, tm, reg):
    u = ue[...]
    v = ie[...]
    gm[...] = jax.nn.sigmoid(jnp.sum(u * v, axis=1))

    utv = _realign(uteg[...], uoff[...])
    itv = _realign(iteg[...], ioff[...])

    def mlp(t):
        h = jnp.maximum(jnp.dot(t, w1[...], preferred_element_type=jnp.float32) + b1[...], 0.0)
        return jnp.dot(h, w2[...], preferred_element_type=jnp.float32) + b2[...]

    ut = utr[...]
    it_ = itr[...]
    td = (jnp.sum(utv[:, :_TT] * ut, axis=1) + jnp.sum(utv[:, _TT:] * mlp(ut), axis=1)
          + jnp.sum(itv[:, :_TT] * it_, axis=1) + jnp.sum(itv[:, _TT:] * mlp(it_), axis=1))
    tm[...] = jax.nn.sigmoid(td)

    n = pl.num_programs(0) * u.shape[0]
    part = (jnp.sum(u * u) + jnp.sum(v * v) + jnp.sum(utv * utv) + jnp.sum(itv * itv)) * (0.5 / n)

    @pl.when(pl.program_id(0) == 0)
    def _():
        reg[...] = jnp.zeros_like(reg)

    reg[...] += part


def _tc_stage(ue, ie, uteg, iteg, uoff, ioff, user_trends, item_trends,
              w1, b1, w2, b2, interpret=False):
    b = ue.shape[0]
    bl = 2048
    grid = (b // bl,)
    return pl.pallas_call(
        _tc_body,
        grid=grid,
        in_specs=[
            pl.BlockSpec((bl, _D), lambda i: (i, 0)),
            pl.BlockSpec((bl, _D), lambda i: (i, 0)),
            pl.BlockSpec((bl, _WIN), lambda i: (i, 0)),
            pl.BlockSpec((bl, _WIN), lambda i: (i, 0)),
            pl.BlockSpec((bl, 1), lambda i: (i, 0)),
            pl.BlockSpec((bl, 1), lambda i: (i, 0)),
            pl.BlockSpec((bl, _TT), lambda i: (i, 0)),
            pl.BlockSpec((bl, _TT), lambda i: (i, 0)),
            pl.BlockSpec((_TT, _D), lambda i: (0, 0)),
            pl.BlockSpec((1, _D), lambda i: (0, 0)),
            pl.BlockSpec((_D, _MT - _TT), lambda i: (0, 0)),
            pl.BlockSpec((1, _MT - _TT), lambda i: (0, 0)),
        ],
        out_specs=[
            pl.BlockSpec((bl,), lambda i: (i,)),
            pl.BlockSpec((bl,), lambda i: (i,)),
            pl.BlockSpec((1, 1), lambda i: (0, 0)),
        ],
        out_shape=[
            jax.ShapeDtypeStruct((b,), jnp.float32),
            jax.ShapeDtypeStruct((b,), jnp.float32),
            jax.ShapeDtypeStruct((1, 1), jnp.float32),
        ],
        interpret=interpret,
    )(ue, ie, uteg, iteg, uoff, ioff, user_trends, item_trends,
      w1, b1.reshape(1, -1), w2, b2.reshape(1, -1))


def _prep_granules(idx):
    g0 = (idx * _MT) // _GR
    g4 = g0[:, None] + jnp.arange(_NG, dtype=jnp.int32)[None, :]
    off = ((idx * _MT) % _GR).reshape(-1, 1)
    return g4.reshape(-1, _CH), off


def kernel(user_indices, item_indices, time_diffs, user_trends, item_trends,
           user_table, item_table, user_time_table, item_time_table, W1, b1, W2, b2):
    del time_diffs
    b = user_indices.shape[0]
    ui = user_indices.astype(jnp.int32)
    ii = item_indices.astype(jnp.int32)
    ug2, uoff = _prep_granules(ui)
    ig2, ioff = _prep_granules(ii)
    utt_g = user_time_table.reshape(-1, _GR)
    itt_g = item_time_table.reshape(-1, _GR)
    ue, ie, uteg, iteg = _sc_gather(user_table, item_table, utt_g, itt_g,
                                    ui.reshape(b // _CH, _CH),
                                    ii.reshape(b // _CH, _CH), ug2, ig2)
    gm, tm, reg = _tc_stage(ue, ie, uteg, iteg, uoff, ioff,
                            user_trends, item_trends, W1, b1, W2, b2)
    return gm, tm, reg[0, 0]


# speedup vs baseline: 1.0108x; 1.0108x over previous
import functools

import jax
import jax.numpy as jnp
from jax import lax
from jax.experimental import pallas as pl
from jax.experimental.pallas import tpu as pltpu
from jax.experimental.pallas import tpu_sc as plsc

_D = 32
_TT = 20
_MT = 50

_NC, _NS = 2, 16
_NW = _NC * _NS
_LAG = 4


def _sc_gather(user_table, item_table, user_time_table, item_time_table, ui, ii):
    b = ui.shape[0]
    rows_w = b // _NW
    niter = rows_w // 16
    mesh = plsc.VectorSubcoreMesh(core_axis_name="c", subcore_axis_name="s")

    @functools.partial(
        pl.kernel,
        out_type=[
            jax.ShapeDtypeStruct((b, _D), jnp.float32),
            jax.ShapeDtypeStruct((b, _D), jnp.float32),
            jax.ShapeDtypeStruct((b, _MT), jnp.float32),
            jax.ShapeDtypeStruct((b, _MT), jnp.float32),
        ],
        mesh=mesh,
        scratch_types=[
            pltpu.VMEM((rows_w,), jnp.int32),
            pltpu.VMEM((rows_w,), jnp.int32),
            pltpu.SemaphoreType.DMA,
        ],
    )
    def k(ut_hbm, it_hbm, utt_hbm, itt_hbm, ui_hbm, ii_hbm,
          ue_hbm, ie_hbm, ute_hbm, ite_hbm, uidx_v, iidx_v, sem):
        wid = lax.axis_index("s") * _NC + lax.axis_index("c")
        base = wid * rows_w
        pltpu.sync_copy(ui_hbm.at[pl.ds(base, rows_w)], uidx_v)
        pltpu.sync_copy(ii_hbm.at[pl.ds(base, rows_w)], iidx_v)

        def drain16(r0):
            for j in range(16):
                r = r0 + j
                pltpu.make_async_copy(ut_hbm.at[pl.ds(0, 1)],
                                      ue_hbm.at[pl.ds(r, 1)], sem).wait()
                pltpu.make_async_copy(it_hbm.at[pl.ds(0, 1)],
                                      ie_hbm.at[pl.ds(r, 1)], sem).wait()
                pltpu.make_async_copy(utt_hbm.at[pl.ds(0, 1)],
                                      ute_hbm.at[pl.ds(r, 1)], sem).wait()
                pltpu.make_async_copy(itt_hbm.at[pl.ds(0, 1)],
                                      ite_hbm.at[pl.ds(r, 1)], sem).wait()

        def body(t, _):
            vu = uidx_v[pl.ds(t * 16, 16)]
            vi = iidx_v[pl.ds(t * 16, 16)]
            for j in range(16):
                r = base + t * 16 + j
                iu = vu[j]
                iv = vi[j]
                pltpu.async_copy(ut_hbm.at[pl.ds(iu, 1)], ue_hbm.at[pl.ds(r, 1)], sem)
                pltpu.async_copy(it_hbm.at[pl.ds(iv, 1)], ie_hbm.at[pl.ds(r, 1)], sem)
                pltpu.async_copy(utt_hbm.at[pl.ds(iu, 1)], ute_hbm.at[pl.ds(r, 1)], sem)
                pltpu.async_copy(itt_hbm.at[pl.ds(iv, 1)], ite_hbm.at[pl.ds(r, 1)], sem)

            @pl.when(t >= _LAG)
            def _():
                drain16(base + (t - _LAG) * 16)

            return 0

        lax.fori_loop(0, niter, body, 0)

        def tail(t, _):
            drain16(base + (niter - _LAG + t) * 16)
            return 0

        lax.fori_loop(0, _LAG, tail, 0)

    return k(user_table, item_table, user_time_table, item_time_table, ui, ii)


def _tc_body(ue, ie, ute, ite, utr, itr, w1, b1, w2, b2, gm, tm, reg):
    u = ue[...]
    v = ie[...]
    gm[...] = jax.nn.sigmoid(jnp.sum(u * v, axis=1))

    def mlp(t):
        h = jnp.maximum(jnp.dot(t, w1[...], preferred_element_type=jnp.float32) + b1[...], 0.0)
        return jnp.dot(h, w2[...], preferred_element_type=jnp.float32) + b2[...]

    utv = ute[...]
    itv = ite[...]
    ut = utr[...]
    it_ = itr[...]
    td = (jnp.sum(utv[:, :_TT] * ut, axis=1) + jnp.sum(utv[:, _TT:] * mlp(ut), axis=1)
          + jnp.sum(itv[:, :_TT] * it_, axis=1) + jnp.sum(itv[:, _TT:] * mlp(it_), axis=1))
    tm[...] = jax.nn.sigmoid(td)

    n = pl.num_programs(0) * u.shape[0]
    part = (jnp.sum(u * u) + jnp.sum(v * v) + jnp.sum(utv * utv) + jnp.sum(itv * itv)) * (0.5 / n)

    @pl.when(pl.program_id(0) == 0)
    def _():
        reg[...] = jnp.zeros_like(reg)

    reg[...] += part


def _tc_stage(ue, ie, ute, ite, user_trends, item_trends, w1, b1, w2, b2,
              interpret=False):
    b = ue.shape[0]
    bl = 2048
    grid = (b // bl,)
    return pl.pallas_call(
        _tc_body,
        grid=grid,
        in_specs=[
            pl.BlockSpec((bl, _D), lambda i: (i, 0)),
            pl.BlockSpec((bl, _D), lambda i: (i, 0)),
            pl.BlockSpec((bl, _MT), lambda i: (i, 0)),
            pl.BlockSpec((bl, _MT), lambda i: (i, 0)),
            pl.BlockSpec((bl, _TT), lambda i: (i, 0)),
            pl.BlockSpec((bl, _TT), lambda i: (i, 0)),
            pl.BlockSpec((_TT, _D), lambda i: (0, 0)),
            pl.BlockSpec((1, _D), lambda i: (0, 0)),
            pl.BlockSpec((_D, _MT - _TT), lambda i: (0, 0)),
            pl.BlockSpec((1, _MT - _TT), lambda i: (0, 0)),
        ],
        out_specs=[
            pl.BlockSpec((bl,), lambda i: (i,)),
            pl.BlockSpec((bl,), lambda i: (i,)),
            pl.BlockSpec((1, 1), lambda i: (0, 0)),
        ],
        out_shape=[
            jax.ShapeDtypeStruct((b,), jnp.float32),
            jax.ShapeDtypeStruct((b,), jnp.float32),
            jax.ShapeDtypeStruct((1, 1), jnp.float32),
        ],
        interpret=interpret,
    )(ue, ie, ute, ite, user_trends, item_trends,
      w1, b1.reshape(1, -1), w2, b2.reshape(1, -1))


def kernel(user_indices, item_indices, time_diffs, user_trends, item_trends,
           user_table, item_table, user_time_table, item_time_table, W1, b1, W2, b2):
    del time_diffs
    ui = user_indices.astype(jnp.int32)
    ii = item_indices.astype(jnp.int32)
    ue, ie, ute, ite = _sc_gather(user_table, item_table,
                                  user_time_table, item_time_table, ui, ii)
    gm, tm, reg = _tc_stage(ue, ie, ute, ite, user_trends, item_trends, W1, b1, W2, b2)
    return gm, tm, reg[0, 0]


# speedup vs baseline: 2.3135x; 2.2888x over previous
import functools

import jax
import jax.numpy as jnp
from jax import lax
from jax.experimental import pallas as pl
from jax.experimental.pallas import tpu as pltpu
from jax.experimental.pallas import tpu_sc as plsc

_D = 32
_TT = 20
_MT = 50

_NC, _NS = 2, 16
_NW = _NC * _NS
_LAG = 4


def _sc_gather(user_table, item_table, user_time_table, item_time_table, ui, ii):
    b = ui.shape[0]
    rows_w = b // _NW
    ch = 128
    nchunk = rows_w // ch
    niter = ch // 16
    mesh = plsc.VectorSubcoreMesh(core_axis_name="c", subcore_axis_name="s")

    @functools.partial(
        pl.kernel,
        out_type=[
            jax.ShapeDtypeStruct((b, _D), jnp.float32),
            jax.ShapeDtypeStruct((b, _D), jnp.float32),
            jax.ShapeDtypeStruct((b, _MT), jnp.float32),
            jax.ShapeDtypeStruct((b, _MT), jnp.float32),
        ],
        mesh=mesh,
        scratch_types=[
            pltpu.VMEM((rows_w,), jnp.int32),
            pltpu.VMEM((rows_w,), jnp.int32),
            pltpu.VMEM((ch, _D), jnp.float32),
            pltpu.VMEM((ch, _D), jnp.float32),
            pltpu.VMEM((ch, _MT), jnp.float32),
            pltpu.VMEM((ch, _MT), jnp.float32),
            pltpu.SemaphoreType.DMA,
        ],
    )
    def k(ut_hbm, it_hbm, utt_hbm, itt_hbm, ui_hbm, ii_hbm,
          ue_hbm, ie_hbm, ute_hbm, ite_hbm,
          uidx_v, iidx_v, ue_v, ie_v, ute_v, ite_v, sem):
        wid = lax.axis_index("s") * _NC + lax.axis_index("c")
        base = wid * rows_w
        pltpu.sync_copy(ui_hbm.at[pl.ds(base, rows_w)], uidx_v)
        pltpu.sync_copy(ii_hbm.at[pl.ds(base, rows_w)], iidx_v)

        def drain16(r0):
            for j in range(16):
                r = r0 + j
                pltpu.make_async_copy(ut_hbm.at[pl.ds(0, 1)],
                                      ue_v.at[pl.ds(r, 1)], sem).wait()
                pltpu.make_async_copy(it_hbm.at[pl.ds(0, 1)],
                                      ie_v.at[pl.ds(r, 1)], sem).wait()
                pltpu.make_async_copy(utt_hbm.at[pl.ds(0, 1)],
                                      ute_v.at[pl.ds(r, 1)], sem).wait()
                pltpu.make_async_copy(itt_hbm.at[pl.ds(0, 1)],
                                      ite_v.at[pl.ds(r, 1)], sem).wait()

        def chunk(c, _):
            def body(t, _unused):
                vu = uidx_v[pl.ds(c * ch + t * 16, 16)]
                vi = iidx_v[pl.ds(c * ch + t * 16, 16)]
                for j in range(16):
                    r = t * 16 + j
                    iu = vu[j]
                    iv = vi[j]
                    pltpu.async_copy(ut_hbm.at[pl.ds(iu, 1)], ue_v.at[pl.ds(r, 1)], sem)
                    pltpu.async_copy(it_hbm.at[pl.ds(iv, 1)], ie_v.at[pl.ds(r, 1)], sem)
                    pltpu.async_copy(utt_hbm.at[pl.ds(iu, 1)], ute_v.at[pl.ds(r, 1)], sem)
                    pltpu.async_copy(itt_hbm.at[pl.ds(iv, 1)], ite_v.at[pl.ds(r, 1)], sem)

                @pl.when(t >= _LAG)
                def _():
                    drain16((t - _LAG) * 16)

                return 0

            lax.fori_loop(0, niter, body, 0)

            def tail(t, _unused):
                drain16((niter - _LAG + t) * 16)
                return 0

            lax.fori_loop(0, _LAG, tail, 0)

            cbase = base + c * ch
            pltpu.sync_copy(ue_v, ue_hbm.at[pl.ds(cbase, ch)])
            pltpu.sync_copy(ie_v, ie_hbm.at[pl.ds(cbase, ch)])
            pltpu.sync_copy(ute_v, ute_hbm.at[pl.ds(cbase, ch)])
            pltpu.sync_copy(ite_v, ite_hbm.at[pl.ds(cbase, ch)])
            return 0

        lax.fori_loop(0, nchunk, chunk, 0)

    return k(user_table, item_table, user_time_table, item_time_table, ui, ii)


def _tc_body(ue, ie, ute, ite, utr, itr, w1, b1, w2, b2, gm, tm, reg):
    u = ue[...]
    v = ie[...]
    gm[...] = jax.nn.sigmoid(jnp.sum(u * v, axis=1))

    def mlp(t):
        h = jnp.maximum(jnp.dot(t, w1[...], preferred_element_type=jnp.float32) + b1[...], 0.0)
        return jnp.dot(h, w2[...], preferred_element_type=jnp.float32) + b2[...]

    utv = ute[...]
    itv = ite[...]
    ut = utr[...]
    it_ = itr[...]
    td = (jnp.sum(utv[:, :_TT] * ut, axis=1) + jnp.sum(utv[:, _TT:] * mlp(ut), axis=1)
          + jnp.sum(itv[:, :_TT] * it_, axis=1) + jnp.sum(itv[:, _TT:] * mlp(it_), axis=1))
    tm[...] = jax.nn.sigmoid(td)

    n = pl.num_programs(0) * u.shape[0]
    part = (jnp.sum(u * u) + jnp.sum(v * v) + jnp.sum(utv * utv) + jnp.sum(itv * itv)) * (0.5 / n)

    @pl.when(pl.program_id(0) == 0)
    def _():
        reg[...] = jnp.zeros_like(reg)

    reg[...] += part


def _tc_stage(ue, ie, ute, ite, user_trends, item_trends, w1, b1, w2, b2,
              interpret=False):
    b = ue.shape[0]
    bl = 2048
    grid = (b // bl,)
    return pl.pallas_call(
        _tc_body,
        grid=grid,
        in_specs=[
            pl.BlockSpec((bl, _D), lambda i: (i, 0)),
            pl.BlockSpec((bl, _D), lambda i: (i, 0)),
            pl.BlockSpec((bl, _MT), lambda i: (i, 0)),
            pl.BlockSpec((bl, _MT), lambda i: (i, 0)),
            pl.BlockSpec((bl, _TT), lambda i: (i, 0)),
            pl.BlockSpec((bl, _TT), lambda i: (i, 0)),
            pl.BlockSpec((_TT, _D), lambda i: (0, 0)),
            pl.BlockSpec((1, _D), lambda i: (0, 0)),
            pl.BlockSpec((_D, _MT - _TT), lambda i: (0, 0)),
            pl.BlockSpec((1, _MT - _TT), lambda i: (0, 0)),
        ],
        out_specs=[
            pl.BlockSpec((bl,), lambda i: (i,)),
            pl.BlockSpec((bl,), lambda i: (i,)),
            pl.BlockSpec((1, 1), lambda i: (0, 0)),
        ],
        out_shape=[
            jax.ShapeDtypeStruct((b,), jnp.float32),
            jax.ShapeDtypeStruct((b,), jnp.float32),
            jax.ShapeDtypeStruct((1, 1), jnp.float32),
        ],
        interpret=interpret,
    )(ue, ie, ute, ite, user_trends, item_trends,
      w1, b1.reshape(1, -1), w2, b2.reshape(1, -1))


def kernel(user_indices, item_indices, time_diffs, user_trends, item_trends,
           user_table, item_table, user_time_table, item_time_table, W1, b1, W2, b2):
    del time_diffs
    ui = user_indices.astype(jnp.int32)
    ii = item_indices.astype(jnp.int32)
    ue, ie, ute, ite = _sc_gather(user_table, item_table,
                                  user_time_table, item_time_table, ui, ii)
    gm, tm, reg = _tc_stage(ue, ie, ute, ite, user_trends, item_trends, W1, b1, W2, b2)
    return gm, tm, reg[0, 0]
